# baseline (speedup 1.0000x reference)
import functools

import numpy as np

import jax
import jax.numpy as jnp
from jax import lax
from jax.experimental import pallas as pl
from jax.experimental.pallas import tpu as pltpu
from jax.experimental.pallas import tpu_sc as plsc

B = 4096
D = 64
D2 = 2 * D
LQ = 20
LR = 50
NEG = 5
T = 12

NC = 2
NS = 16
NW = NC * NS
GCH = 128
SLAB = 640

OFF_W = 0
OFF_WN = OFF_W + B * LR
OFF_Q = OFF_WN + B * LR
OFF_IN = OFF_Q + B * LQ
OFF_U = OFF_IN + B * NEG
OFF_I = OFF_U + B
G_ROWS = OFF_I + B

_EPS_KEYS = [(0x6D3E048F, 0x1022172D), (0x03D7B32D, 0xADD083F4),
             (0x92FB20EA, 0x0F38D913), (0xBAD56946, 0x354BA891),
             (0xB013AEE3, 0xC34EDDF6)]


def _threefry2x32_np(k1, k2, x0, x1):
    def rotl(x, d):
        return ((x << np.uint32(d)) | (x >> np.uint32(32 - d))).astype(
            np.uint32)

    ks = [np.uint32(k1), np.uint32(k2),
          np.uint32(k1) ^ np.uint32(k2) ^ np.uint32(0x1BD11BDA)]
    x = [x0.astype(np.uint32) + ks[0], x1.astype(np.uint32) + ks[1]]

    def rounds(rs):
        for r in rs:
            x[0] = (x[0] + x[1]).astype(np.uint32)
            x[1] = x[0] ^ rotl(x[1], r)

    rounds((13, 15, 26, 6)); x[0] += ks[1]; x[1] += ks[2] + np.uint32(1)
    rounds((17, 29, 16, 24)); x[0] += ks[2]; x[1] += ks[0] + np.uint32(2)
    rounds((13, 15, 26, 6)); x[0] += ks[0]; x[1] += ks[1] + np.uint32(3)
    rounds((17, 29, 16, 24)); x[0] += ks[1]; x[1] += ks[2] + np.uint32(4)
    rounds((13, 15, 26, 6)); x[0] += ks[2]; x[1] += ks[0] + np.uint32(5)
    return x[0].astype(np.uint32), x[1].astype(np.uint32)


def _erfinv_np(x):
    x = x.astype(np.float64)
    w = -np.log1p(-x * x)
    wa = w - 2.5
    pa = 2.81022636e-08
    for c in (3.43273939e-07, -3.5233877e-06, -4.39150654e-06, 0.00021858087,
              -0.00125372503, -0.00417768164, 0.246640727, 1.50140941):
        pa = c + pa * wa
    wb = np.sqrt(np.maximum(w, 5.0)) - 3.0
    pb = -0.000200214257
    for c in (0.000100950558, 0.00134934322, -0.00367342844, 0.00573950773,
              -0.0076224613, 0.00943887047, 1.00167406, 2.83297682):
        pb = c + pb * wb
    return np.where(w < 5.0, pa, pb) * x


def _eps_np(key_idx, n):
    old = np.seterr(over='ignore')
    k1, k2 = _EPS_KEYS[key_idx]
    j = np.arange(n, dtype=np.uint32)
    b1, b2 = _threefry2x32_np(k1, k2, np.zeros(n, np.uint32), j)
    bits = b1 ^ b2
    fb = (bits >> np.uint32(9)) | np.uint32(0x3F800000)
    floats = fb.view(np.float32) - np.float32(1.0)
    lo = np.nextafter(np.float32(-1), np.float32(0), dtype=np.float32)
    hi = np.float32(1.0)
    u = np.maximum(lo, floats * (hi - lo) + lo)
    out = (np.sqrt(2.0) * _erfinv_np(u)).astype(np.float32)
    np.seterr(**old)
    return out.reshape(n // (2 * D), 2 * D)


_EPS = [_eps_np(0, B * D), _eps_np(1, B * D), _eps_np(2, B * NEG * D),
        _eps_np(3, B * LR * D), _eps_np(4, B * LR * D)]


BB = 128


def _perm_np(n, bs):
    p = np.arange(n, dtype=np.int32).reshape(n // bs, bs // 2, 2)
    return np.ascontiguousarray(p.transpose(0, 2, 1)).reshape(n)


_PERM = {(n, bs): _perm_np(n, bs)
         for n, bs in [(B, BB), (B * NEG, BB * NEG), (B * LQ, BB * LQ),
                       (B * LR, BB * LR)]}


def _sc_gather_build(interpret=False):
    mesh = plsc.VectorSubcoreMesh(
        core_axis_name="c", subcore_axis_name="s", num_cores=NC, num_subcores=NS
    )
    f32 = jnp.float32
    out_type = jax.ShapeDtypeStruct((G_ROWS, D2), f32)
    scratch_types = [
        pltpu.VMEM((SLAB,), jnp.int32),
        pltpu.VMEM((SLAB, D2), f32),
        pltpu.SemaphoreType.DMA,
        pltpu.SemaphoreType.DMA,
    ]

    def body(u_idx, i_idx, in_idx, q_idx, w_idx, wn_idx,
             user_cat, item_cat, word_cat,
             out, idx_v, rows_v, sem, sem_b):
        wid = lax.axis_index("s") * NC + lax.axis_index("c")

        def stream(idx1d, tab, seg, total):
            per_w = total // NW
            slab = per_w if per_w < SLAB else SLAB
            ng = slab // GCH
            n_slab = per_w // slab
            base = wid * per_w

            def do_slab(s, _):
                off = base + s * slab
                pltpu.sync_copy(idx1d.at[pl.ds(off, slab)],
                                idx_v.at[pl.ds(0, slab)])
                copies = []
                for j in range(ng):
                    copies.append(pltpu.async_copy(
                        tab.at[idx_v.at[pl.ds(j * GCH, GCH)]],
                        rows_v.at[pl.ds(j * GCH, GCH)], sem))
                for c in copies:
                    c.wait()
                pltpu.sync_copy(rows_v.at[pl.ds(0, slab)],
                                out.at[pl.ds(seg + off, slab)])
                return 0

            if n_slab == 1:
                do_slab(0, 0)
            else:
                lax.fori_loop(0, n_slab, do_slab, 0, unroll=False)

        def stream2(idx1d, tab, seg, total):
            per_w = total // NW
            base = wid * per_w
            n2 = per_w // (2 * GCH)

            def do2(s2, _):
                o_a = base + s2 * (2 * GCH)
                o_b = o_a + GCH
                pltpu.sync_copy(idx1d.at[pl.ds(o_a, GCH)],
                                idx_v.at[pl.ds(0, GCH)])
                ca = pltpu.async_copy(
                    tab.at[idx_v.at[pl.ds(0, GCH)]],
                    rows_v.at[pl.ds(0, GCH)], sem)
                pltpu.sync_copy(idx1d.at[pl.ds(o_b, GCH)],
                                idx_v.at[pl.ds(GCH, GCH)])
                cb = pltpu.async_copy(
                    tab.at[idx_v.at[pl.ds(GCH, GCH)]],
                    rows_v.at[pl.ds(GCH, GCH)], sem_b)
                ca.wait()
                pltpu.sync_copy(rows_v.at[pl.ds(0, GCH)],
                                out.at[pl.ds(seg + o_a, GCH)])
                cb.wait()
                pltpu.sync_copy(rows_v.at[pl.ds(GCH, GCH)],
                                out.at[pl.ds(seg + o_b, GCH)])
                return 0

            lax.fori_loop(0, n2, do2, 0, unroll=False)

        stream2(w_idx, word_cat, OFF_W, B * LR)
        stream2(wn_idx, word_cat, OFF_WN, B * LR)
        stream2(q_idx, word_cat, OFF_Q, B * LQ)
        stream(in_idx, item_cat, OFF_IN, B * NEG)
        stream(u_idx, user_cat, OFF_U, B)
        stream(i_idx, item_cat, OFF_I, B)

    return pl.kernel(
        body, out_type=out_type, mesh=mesh, scratch_types=scratch_types,
        compiler_params=pltpu.CompilerParams(use_tc_tiling_on_sc=True),
        interpret=interpret)


GRID = B // BB


def _halfsel_onehot(npairs, n, odd):
    r = 2 * lax.broadcasted_iota(jnp.int32, (npairs, BB), 0) + odd
    bn = n * lax.broadcasted_iota(jnp.int32, (npairs, BB), 1)
    d = r - bn
    return ((d >= 0) & (d < n)).astype(jnp.float32)


def _tc_finish_body(g_w, g_wn, g_q, g_in, g_u, g_i,
                    times2d, qlen2d, time_emb,
                    Cu, Ci, Cw,
                    Tmu, Tsu, Tmi, Tsi, Tmw, Tsw, Wq,
                    bmu, bsu, bmi, bsi, bmw, bsw, bq,
                    e_u, e_ip, e_in, e_w, e_wn,
                    o_q, o_user, o_item, o_ineg, o_w, o_wn):
    f32 = jnp.float32
    dot = functools.partial(jnp.dot, preferred_element_type=f32)
    cat1 = functools.partial(jnp.concatenate, axis=1)

    tp1 = times2d[...] + 1
    oh = (lax.broadcasted_iota(jnp.int32, (BB, T), 1) == tp1).astype(f32)
    tl = dot(oh, time_emb[...])

    def sample(g, Cat, Tm, Ts, bm, bs, eps, n):
        tlc = cat1([dot(tl, Tm[...]) + bm[...], dot(tl, Ts[...]) + bs[...]])
        R = (BB * n) // 2
        x = g[...]
        y_e = dot(x[:R], Cat[...])
        y_o = dot(x[R:], Cat[...])
        y_e = y_e + dot(_halfsel_onehot(R, n, 0), tlc)
        y_o = y_o + dot(_halfsel_onehot(R, n, 1), tlc)
        mean2 = cat1([y_e[:, :D], y_o[:, :D]])
        spre2 = cat1([y_e[:, D:], y_o[:, D:]])
        return mean2 + jnp.exp(0.5 * spre2) * eps[...]

    o_user[...] = sample(g_u, Cu, Tmu, Tsu, bmu, bsu, e_u, 1)
    o_item[...] = sample(g_i, Ci, Tmi, Tsi, bmi, bsi, e_ip, 1)
    o_ineg[...] = sample(g_in, Ci, Tmi, Tsi, bmi, bsi, e_in, NEG)
    o_w[...] = sample(g_w, Cw, Tmw, Tsw, bmw, bsw, e_w, LR)
    o_wn[...] = sample(g_wn, Cw, Tmw, Tsw, bmw, bsw, e_wn, LR)

    qlen = qlen2d[...]
    QP = (BB * LQ) // 2
    xq = g_q[...]
    r2 = 2 * lax.broadcasted_iota(jnp.int32, (BB, QP), 1)
    bi = LQ * lax.broadcasted_iota(jnp.int32, (BB, QP), 0)
    de = r2 - bi
    do = de + 1
    me = ((de >= 0) & (de < qlen)).astype(f32)
    mo = ((do >= 0) & (do < qlen)).astype(f32)
    qsum = dot(me, xq[:QP, :D]) + dot(mo, xq[QP:, :D])
    qmean = qsum / qlen.astype(f32)
    o_q[...] = jnp.tanh(dot(qmean, Wq[...]) + bq[...])


def _tc_finish_build(interpret=False):
    f32 = jnp.float32

    def seg(rows_per_blk, off):
        blk_off = off // rows_per_blk
        return pl.BlockSpec((rows_per_blk, D2),
                            lambda i, o=blk_off: (i + o, 0))

    def full(shape):
        nd = len(shape)
        return pl.BlockSpec(shape, lambda i: (0,) * nd)

    def rows(r, d=D):
        return pl.BlockSpec((r, d), lambda i: (i, 0))

    WBLK = BB * LR
    in_specs = [
        seg(WBLK, OFF_W), seg(WBLK, OFF_WN), seg(BB * LQ, OFF_Q),
        seg(BB * NEG, OFF_IN), seg(BB, OFF_U), seg(BB, OFF_I),
        pl.BlockSpec((BB, 1), lambda i: (i, 0)),
        pl.BlockSpec((BB, 1), lambda i: (i, 0)),
        full((T, D)),
        full((D2, D2)), full((D2, D2)), full((D2, D2)),
        full((D, D)), full((D, D)), full((D, D)),
        full((D, D)), full((D, D)), full((D, D)), full((D, D)),
        full((1, D)), full((1, D)), full((1, D)), full((1, D)),
        full((1, D)), full((1, D)), full((1, D)),
        rows(BB // 2, D2), rows(BB // 2, D2), rows(BB * NEG // 2, D2),
        rows(WBLK // 2, D2), rows(WBLK // 2, D2),
    ]
    out_specs = [rows(BB), rows(BB // 2, D2), rows(BB // 2, D2),
                 rows(BB * NEG // 2, D2), rows(WBLK // 2, D2),
                 rows(WBLK // 2, D2)]
    out_shape = [
        jax.ShapeDtypeStruct((B, D), f32),
        jax.ShapeDtypeStruct((B // 2, D2), f32),
        jax.ShapeDtypeStruct((B // 2, D2), f32),
        jax.ShapeDtypeStruct((B * NEG // 2, D2), f32),
        jax.ShapeDtypeStruct((B * LR // 2, D2), f32),
        jax.ShapeDtypeStruct((B * LR // 2, D2), f32),
    ]
    return pl.pallas_call(
        _tc_finish_body, grid=(GRID,), in_specs=in_specs,
        out_specs=out_specs, out_shape=out_shape, interpret=interpret)


def _blkdiag2(Wm, Ws):
    z = jnp.zeros((D, D), Wm.dtype)
    return jnp.concatenate([jnp.concatenate([Wm, z], 1),
                            jnp.concatenate([z, Ws], 1)], 0)


def _run(interpret_sc, interpret_tc,
         time_emb, user_mean_t, user_std_t, item_mean_t, item_std_t,
         word_mean_t, word_std_t,
         W_t2m_u, b_t2m_u, W_t2s_u, b_t2s_u, W_t2m_i, b_t2m_i,
         W_t2s_i, b_t2s_i, W_t2m_w, b_t2m_w, W_t2s_w, b_t2s_w, W_q, b_q,
         user, item_pos, query, query_len, word, word_len, times,
         items_neg, word_neg):
    i32 = jnp.int32

    def idx1(a, bs):
        f = a.reshape(-1).astype(i32)
        return jnp.take(f, _PERM[f.shape[0], bs])

    cat = lambda m, s: jnp.concatenate([m, s], axis=1)
    g = _sc_gather_build(interpret_sc)(
        idx1(user, BB), idx1(item_pos, BB), idx1(items_neg, BB * NEG),
        idx1(query, BB * LQ), idx1(word, BB * LR), idx1(word_neg, BB * LR),
        cat(user_mean_t, user_std_t), cat(item_mean_t, item_std_t),
        cat(word_mean_t, word_std_t))

    outs = _tc_finish_build(interpret_tc)(
        g, g, g, g, g, g,
        times.reshape(B, 1).astype(i32), query_len.reshape(B, 1).astype(i32),
        time_emb,
        _blkdiag2(W_t2m_u[:D], W_t2s_u[:D]),
        _blkdiag2(W_t2m_i[:D], W_t2s_i[:D]),
        _blkdiag2(W_t2m_w[:D], W_t2s_w[:D]),
        W_t2m_u[D:], W_t2s_u[D:], W_t2m_i[D:], W_t2s_i[D:],
        W_t2m_w[D:], W_t2s_w[D:], W_q,
        b_t2m_u.reshape(1, D), b_t2s_u.reshape(1, D),
        b_t2m_i.reshape(1, D), b_t2s_i.reshape(1, D),
        b_t2m_w.reshape(1, D), b_t2s_w.reshape(1, D), b_q.reshape(1, D),
        _EPS[0], _EPS[1], _EPS[2], _EPS[3], _EPS[4])
    q, user_s, item_s, ineg_s, w_s, wn_s = outs
    return jnp.concatenate([q.reshape(-1), user_s.reshape(-1),
                            item_s.reshape(-1), ineg_s.reshape(-1),
                            w_s.reshape(-1), wn_s.reshape(-1)])


def kernel(time_emb, user_mean_t, user_std_t, item_mean_t, item_std_t,
           word_mean_t, word_std_t,
           W_t2m_u, b_t2m_u, W_t2s_u, b_t2s_u, W_t2m_i, b_t2m_i,
           W_t2s_i, b_t2s_i, W_t2m_w, b_t2m_w, W_t2s_w, b_t2s_w, W_q, b_q,
           user, item_pos, query, query_len, word, word_len, times,
           items_neg, word_neg):
    return _run(False, False,
                time_emb, user_mean_t, user_std_t, item_mean_t, item_std_t,
                word_mean_t, word_std_t,
                W_t2m_u, b_t2m_u, W_t2s_u, b_t2s_u, W_t2m_i, b_t2m_i,
                W_t2s_i, b_t2s_i, W_t2m_w, b_t2m_w, W_t2s_w, b_t2s_w,
                W_q, b_q,
                user, item_pos, query, query_len, word, word_len, times,
                items_neg, word_neg)


# speedup vs baseline: 1.0378x; 1.0378x over previous
import functools

import numpy as np

import jax
import jax.numpy as jnp
from jax import lax
from jax.experimental import pallas as pl
from jax.experimental.pallas import tpu as pltpu
from jax.experimental.pallas import tpu_sc as plsc

B = 4096
D = 64
D2 = 2 * D
LQ = 20
LR = 50
NEG = 5
T = 12

NC = 2
NS = 16
NW = NC * NS
GCH = 128
SLAB = 640

OFF_W = 0
OFF_WN = OFF_W + B * LR
OFF_Q = OFF_WN + B * LR
OFF_IN = OFF_Q + B * LQ
OFF_U = OFF_IN + B * NEG
OFF_I = OFF_U + B
G_ROWS = OFF_I + B

_EPS_KEYS = [(0x6D3E048F, 0x1022172D), (0x03D7B32D, 0xADD083F4),
             (0x92FB20EA, 0x0F38D913), (0xBAD56946, 0x354BA891),
             (0xB013AEE3, 0xC34EDDF6)]


def _threefry2x32_np(k1, k2, x0, x1):
    def rotl(x, d):
        return ((x << np.uint32(d)) | (x >> np.uint32(32 - d))).astype(
            np.uint32)

    ks = [np.uint32(k1), np.uint32(k2),
          np.uint32(k1) ^ np.uint32(k2) ^ np.uint32(0x1BD11BDA)]
    x = [x0.astype(np.uint32) + ks[0], x1.astype(np.uint32) + ks[1]]

    def rounds(rs):
        for r in rs:
            x[0] = (x[0] + x[1]).astype(np.uint32)
            x[1] = x[0] ^ rotl(x[1], r)

    rounds((13, 15, 26, 6)); x[0] += ks[1]; x[1] += ks[2] + np.uint32(1)
    rounds((17, 29, 16, 24)); x[0] += ks[2]; x[1] += ks[0] + np.uint32(2)
    rounds((13, 15, 26, 6)); x[0] += ks[0]; x[1] += ks[1] + np.uint32(3)
    rounds((17, 29, 16, 24)); x[0] += ks[1]; x[1] += ks[2] + np.uint32(4)
    rounds((13, 15, 26, 6)); x[0] += ks[2]; x[1] += ks[0] + np.uint32(5)
    return x[0].astype(np.uint32), x[1].astype(np.uint32)


def _erfinv_np(x):
    x = x.astype(np.float64)
    w = -np.log1p(-x * x)
    wa = w - 2.5
    pa = 2.81022636e-08
    for c in (3.43273939e-07, -3.5233877e-06, -4.39150654e-06, 0.00021858087,
              -0.00125372503, -0.00417768164, 0.246640727, 1.50140941):
        pa = c + pa * wa
    wb = np.sqrt(np.maximum(w, 5.0)) - 3.0
    pb = -0.000200214257
    for c in (0.000100950558, 0.00134934322, -0.00367342844, 0.00573950773,
              -0.0076224613, 0.00943887047, 1.00167406, 2.83297682):
        pb = c + pb * wb
    return np.where(w < 5.0, pa, pb) * x


def _eps_np(key_idx, n):
    old = np.seterr(over='ignore')
    k1, k2 = _EPS_KEYS[key_idx]
    j = np.arange(n, dtype=np.uint32)
    b1, b2 = _threefry2x32_np(k1, k2, np.zeros(n, np.uint32), j)
    bits = b1 ^ b2
    fb = (bits >> np.uint32(9)) | np.uint32(0x3F800000)
    floats = fb.view(np.float32) - np.float32(1.0)
    lo = np.nextafter(np.float32(-1), np.float32(0), dtype=np.float32)
    hi = np.float32(1.0)
    u = np.maximum(lo, floats * (hi - lo) + lo)
    out = (np.sqrt(2.0) * _erfinv_np(u)).astype(np.float32)
    np.seterr(**old)
    return out.reshape(n // (2 * D), 2 * D)


_EPS = [_eps_np(0, B * D), _eps_np(1, B * D), _eps_np(2, B * NEG * D),
        _eps_np(3, B * LR * D), _eps_np(4, B * LR * D)]


BB = 128


def _perm_np(n, bs):
    p = np.arange(n, dtype=np.int32).reshape(n // bs, bs // 2, 2)
    return np.ascontiguousarray(p.transpose(0, 2, 1)).reshape(n)


_PERM = {(n, bs): _perm_np(n, bs)
         for n, bs in [(B, BB), (B * NEG, BB * NEG), (B * LQ, BB * LQ),
                       (B * LR, BB * LR)]}


def _sc_gather_build(interpret=False):
    mesh = plsc.VectorSubcoreMesh(
        core_axis_name="c", subcore_axis_name="s", num_cores=NC, num_subcores=NS
    )
    f32 = jnp.float32
    out_type = jax.ShapeDtypeStruct((G_ROWS, D2), f32)
    scratch_types = [
        pltpu.VMEM((SLAB,), jnp.int32),
        pltpu.VMEM((SLAB, D2), f32),
        pltpu.SemaphoreType.DMA,
    ]

    def body(u_idx, i_idx, in_idx, q_idx, w_idx, wn_idx,
             user_cat, item_cat, word_cat,
             out, idx_v, rows_v, sem):
        wid = lax.axis_index("s") * NC + lax.axis_index("c")

        def stream(idx1d, tab, seg, total):
            per_w = total // NW
            slab = per_w if per_w < SLAB else SLAB
            ng = slab // GCH
            n_slab = per_w // slab
            base = wid * per_w

            def do_slab(s, _):
                off = base + s * slab
                pltpu.sync_copy(idx1d.at[pl.ds(off, slab)],
                                idx_v.at[pl.ds(0, slab)])
                copies = []
                for j in range(ng):
                    copies.append(pltpu.async_copy(
                        tab.at[idx_v.at[pl.ds(j * GCH, GCH)]],
                        rows_v.at[pl.ds(j * GCH, GCH)], sem))
                for c in copies:
                    c.wait()
                pltpu.sync_copy(rows_v.at[pl.ds(0, slab)],
                                out.at[pl.ds(seg + off, slab)])
                return 0

            if n_slab == 1:
                do_slab(0, 0)
            else:
                lax.fori_loop(0, n_slab, do_slab, 0, unroll=False)

        stream(w_idx, word_cat, OFF_W, B * LR)
        stream(wn_idx, word_cat, OFF_WN, B * LR)
        stream(q_idx, word_cat, OFF_Q, B * LQ)
        stream(in_idx, item_cat, OFF_IN, B * NEG)
        stream(u_idx, user_cat, OFF_U, B)
        stream(i_idx, item_cat, OFF_I, B)

    return pl.kernel(
        body, out_type=out_type, mesh=mesh, scratch_types=scratch_types,
        compiler_params=pltpu.CompilerParams(use_tc_tiling_on_sc=True),
        interpret=interpret)


GRID = B // BB


def _halfsel_onehot(npairs, n, odd):
    r = 2 * lax.broadcasted_iota(jnp.int32, (npairs, BB), 0) + odd
    bn = n * lax.broadcasted_iota(jnp.int32, (npairs, BB), 1)
    d = r - bn
    return ((d >= 0) & (d < n)).astype(jnp.float32)


def _tc_finish_body(g_w, g_wn, g_q, g_in, g_u, g_i,
                    times2d, qlen2d, time_emb,
                    Cu, Ci, Cw,
                    Tmu, Tsu, Tmi, Tsi, Tmw, Tsw, Wq,
                    bmu, bsu, bmi, bsi, bmw, bsw, bq,
                    e_u, e_ip, e_in, e_w, e_wn,
                    o_q, o_user, o_item, o_ineg, o_w, o_wn):
    f32 = jnp.float32
    dot = functools.partial(jnp.dot, preferred_element_type=f32)
    cat1 = functools.partial(jnp.concatenate, axis=1)

    tp1 = times2d[...] + 1
    oh = (lax.broadcasted_iota(jnp.int32, (BB, T), 1) == tp1).astype(f32)
    tl = dot(oh, time_emb[...])

    def sample(g, Cat, Tm, Ts, bm, bs, eps, n):
        tlc = cat1([dot(tl, Tm[...]) + bm[...], dot(tl, Ts[...]) + bs[...]])
        R = (BB * n) // 2
        x = g[...]
        y_e = dot(x[:R], Cat[...])
        y_o = dot(x[R:], Cat[...])
        y_e = y_e + dot(_halfsel_onehot(R, n, 0), tlc)
        y_o = y_o + dot(_halfsel_onehot(R, n, 1), tlc)
        mean2 = cat1([y_e[:, :D], y_o[:, :D]])
        spre2 = cat1([y_e[:, D:], y_o[:, D:]])
        return mean2 + jnp.exp(0.5 * spre2) * eps[...]

    o_user[...] = sample(g_u, Cu, Tmu, Tsu, bmu, bsu, e_u, 1)
    o_item[...] = sample(g_i, Ci, Tmi, Tsi, bmi, bsi, e_ip, 1)
    o_ineg[...] = sample(g_in, Ci, Tmi, Tsi, bmi, bsi, e_in, NEG)
    o_w[...] = sample(g_w, Cw, Tmw, Tsw, bmw, bsw, e_w, LR)
    o_wn[...] = sample(g_wn, Cw, Tmw, Tsw, bmw, bsw, e_wn, LR)

    qlen = qlen2d[...]
    QP = (BB * LQ) // 2
    xq = g_q[...]
    r2 = 2 * lax.broadcasted_iota(jnp.int32, (BB, QP), 1)
    bi = LQ * lax.broadcasted_iota(jnp.int32, (BB, QP), 0)
    de = r2 - bi
    do = de + 1
    me = ((de >= 0) & (de < qlen)).astype(f32)
    mo = ((do >= 0) & (do < qlen)).astype(f32)
    qsum = dot(me, xq[:QP, :D]) + dot(mo, xq[QP:, :D])
    qmean = qsum / qlen.astype(f32)
    o_q[...] = jnp.tanh(dot(qmean, Wq[...]) + bq[...])


def _tc_finish_build(interpret=False):
    f32 = jnp.float32

    def seg(rows_per_blk, off):
        blk_off = off // rows_per_blk
        return pl.BlockSpec((rows_per_blk, D2),
                            lambda i, o=blk_off: (i + o, 0))

    def full(shape):
        nd = len(shape)
        return pl.BlockSpec(shape, lambda i: (0,) * nd)

    def rows(r, d=D):
        return pl.BlockSpec((r, d), lambda i: (i, 0))

    WBLK = BB * LR
    in_specs = [
        seg(WBLK, OFF_W), seg(WBLK, OFF_WN), seg(BB * LQ, OFF_Q),
        seg(BB * NEG, OFF_IN), seg(BB, OFF_U), seg(BB, OFF_I),
        pl.BlockSpec((BB, 1), lambda i: (i, 0)),
        pl.BlockSpec((BB, 1), lambda i: (i, 0)),
        full((T, D)),
        full((D2, D2)), full((D2, D2)), full((D2, D2)),
        full((D, D)), full((D, D)), full((D, D)),
        full((D, D)), full((D, D)), full((D, D)), full((D, D)),
        full((1, D)), full((1, D)), full((1, D)), full((1, D)),
        full((1, D)), full((1, D)), full((1, D)),
        rows(BB // 2, D2), rows(BB // 2, D2), rows(BB * NEG // 2, D2),
        rows(WBLK // 2, D2), rows(WBLK // 2, D2),
    ]
    out_specs = [rows(BB), rows(BB // 2, D2), rows(BB // 2, D2),
                 rows(BB * NEG // 2, D2), rows(WBLK // 2, D2),
                 rows(WBLK // 2, D2)]
    out_shape = [
        jax.ShapeDtypeStruct((B, D), f32),
        jax.ShapeDtypeStruct((B // 2, D2), f32),
        jax.ShapeDtypeStruct((B // 2, D2), f32),
        jax.ShapeDtypeStruct((B * NEG // 2, D2), f32),
        jax.ShapeDtypeStruct((B * LR // 2, D2), f32),
        jax.ShapeDtypeStruct((B * LR // 2, D2), f32),
    ]
    return pl.pallas_call(
        _tc_finish_body, grid=(GRID,), in_specs=in_specs,
        out_specs=out_specs, out_shape=out_shape, interpret=interpret)


def _blkdiag2(Wm, Ws):
    z = jnp.zeros((D, D), Wm.dtype)
    return jnp.concatenate([jnp.concatenate([Wm, z], 1),
                            jnp.concatenate([z, Ws], 1)], 0)


def _run(interpret_sc, interpret_tc,
         time_emb, user_mean_t, user_std_t, item_mean_t, item_std_t,
         word_mean_t, word_std_t,
         W_t2m_u, b_t2m_u, W_t2s_u, b_t2s_u, W_t2m_i, b_t2m_i,
         W_t2s_i, b_t2s_i, W_t2m_w, b_t2m_w, W_t2s_w, b_t2s_w, W_q, b_q,
         user, item_pos, query, query_len, word, word_len, times,
         items_neg, word_neg):
    i32 = jnp.int32

    def idx1(a, bs):
        f = a.reshape(-1).astype(i32)
        return jnp.take(f, _PERM[f.shape[0], bs])

    cat = lambda m, s: jnp.concatenate([m, s], axis=1)
    g = _sc_gather_build(interpret_sc)(
        idx1(user, BB), idx1(item_pos, BB), idx1(items_neg, BB * NEG),
        idx1(query, BB * LQ), idx1(word, BB * LR), idx1(word_neg, BB * LR),
        cat(user_mean_t, user_std_t), cat(item_mean_t, item_std_t),
        cat(word_mean_t, word_std_t))

    outs = _tc_finish_build(interpret_tc)(
        g, g, g, g, g, g,
        times.reshape(B, 1).astype(i32), query_len.reshape(B, 1).astype(i32),
        time_emb,
        _blkdiag2(W_t2m_u[:D], W_t2s_u[:D]),
        _blkdiag2(W_t2m_i[:D], W_t2s_i[:D]),
        _blkdiag2(W_t2m_w[:D], W_t2s_w[:D]),
        W_t2m_u[D:], W_t2s_u[D:], W_t2m_i[D:], W_t2s_i[D:],
        W_t2m_w[D:], W_t2s_w[D:], W_q,
        b_t2m_u.reshape(1, D), b_t2s_u.reshape(1, D),
        b_t2m_i.reshape(1, D), b_t2s_i.reshape(1, D),
        b_t2m_w.reshape(1, D), b_t2s_w.reshape(1, D), b_q.reshape(1, D),
        _EPS[0], _EPS[1], _EPS[2], _EPS[3], _EPS[4])
    q, user_s, item_s, ineg_s, w_s, wn_s = outs
    return jnp.concatenate([q.reshape(-1), user_s.reshape(-1),
                            item_s.reshape(-1), ineg_s.reshape(-1),
                            w_s.reshape(-1), wn_s.reshape(-1)])


def kernel(time_emb, user_mean_t, user_std_t, item_mean_t, item_std_t,
           word_mean_t, word_std_t,
           W_t2m_u, b_t2m_u, W_t2s_u, b_t2s_u, W_t2m_i, b_t2m_i,
           W_t2s_i, b_t2s_i, W_t2m_w, b_t2m_w, W_t2s_w, b_t2s_w, W_q, b_q,
           user, item_pos, query, query_len, word, word_len, times,
           items_neg, word_neg):
    return _run(False, False,
                time_emb, user_mean_t, user_std_t, item_mean_t, item_std_t,
                word_mean_t, word_std_t,
                W_t2m_u, b_t2m_u, W_t2s_u, b_t2s_u, W_t2m_i, b_t2m_i,
                W_t2s_i, b_t2s_i, W_t2m_w, b_t2m_w, W_t2s_w, b_t2s_w,
                W_q, b_q,
                user, item_pos, query, query_len, word, word_len, times,
                items_neg, word_neg)
